# baseline (speedup 1.0000x reference)
import functools

import jax
import jax.numpy as jnp
from jax.experimental import pallas as pl
from jax.experimental.pallas import tpu as pltpu


def _conv_silu_kernel(x_ref, w_ref, o_ref, eo_ref, p_ref, a_ref, *,
                      k, c1, ri, cq, span, ho, wo):
    b = x_ref.shape[0]
    lanes = x_ref.shape[3]
    ncols = lanes // 2
    rq = ri + 2

    l_io = jax.lax.broadcasted_iota(jnp.int32, (lanes, 256), 0)
    o_io = jax.lax.broadcasted_iota(jnp.int32, (lanes, 256), 1)
    q_o = o_io // cq
    s_o = o_io % cq - 1
    l_want = (q_o // 2) * ncols + 2 * s_o + (q_o % 2)
    valid = (o_io < 4 * cq) & (o_io % cq >= 1) & (s_o < ncols // 2)
    sel = jnp.where(valid & (l_io == l_want), 1.0, 0.0).astype(jnp.bfloat16)

    p_ref[:, :, 0:cq] = jnp.zeros((4, c1, cq), jnp.bfloat16)
    p_ref[:, :, (rq - 1) * cq:rq * cq] = jnp.zeros((4, c1, cq), jnp.bfloat16)

    for n in range(b):
        eo = jnp.dot(x_ref[n].reshape(ri * c1, lanes), sel,
                     preferred_element_type=jnp.float32)
        eo_ref[...] = eo.astype(jnp.bfloat16).reshape(ri, c1, 256)

        for r in range(ri):
            for q in range(4):
                p_ref[q, :, (1 + r) * cq:(2 + r) * cq] = (
                    eo_ref[r, :, q * cq:(q + 1) * cq])

        for kh in range(k):
            for kw in range(k):
                t = kh * k + kw
                q = ((kh + 1) % 2) * 2 + (kw + 1) % 2
                off = (0 if kh == 0 else 1) * cq + (0 if kw == 0 else 1)
                a_ref[t * c1:(t + 1) * c1, n * span:(n + 1) * span] = (
                    p_ref[q, :, off:off + span])

    acc = jnp.dot(w_ref[...], a_ref[...], preferred_element_type=jnp.float32)
    acc = acc * jax.nn.sigmoid(acc)
    for n in range(b):
        for r in range(ho):
            o_ref[n, :, r * wo:(r + 1) * wo] = (
                acc[:, n * span + r * cq:n * span + r * cq + wo])


@functools.partial(jax.jit, static_argnames=("k", "s", "p", "imgs_per_step"))
def _cov_act(x_nchw, weight_oihw, *, k, s, p, imgs_per_step):
    N, C1, H, W = x_nchw.shape
    C2 = weight_oihw.shape[0]
    Ho = (H + 2 * p - k) // s + 1
    Wo = (W + 2 * p - k) // s + 1
    KKC1 = k * k * C1
    RI, CI = H // 2, W // 2
    RQ, CQ = RI + 2, CI + 2
    span = Ho * CQ

    xb = x_nchw.astype(jnp.bfloat16).reshape(N, C1, RI, W, 2)
    xi = jax.lax.bitcast_convert_type(xb, jnp.float32)
    xi = xi.transpose(0, 2, 1, 3)
    x_sr = jax.lax.bitcast_convert_type(xi, jnp.bfloat16)
    x_sr = x_sr.reshape(N, RI, C1, 2 * W)

    w2 = weight_oihw.transpose(0, 2, 3, 1).reshape(C2, KKC1)
    w2 = w2.astype(jnp.bfloat16)

    B = imgs_per_step
    body = functools.partial(_conv_silu_kernel, k=k, c1=C1, ri=RI, cq=CQ,
                             span=span, ho=Ho, wo=Wo)
    cost = pl.CostEstimate(
        flops=2 * C2 * KKC1 * N * span + 2 * N * RI * C1 * 2 * W * 256,
        transcendentals=N * C2 * span,
        bytes_accessed=(N * C1 * H * W + C2 * KKC1) * 2 + N * C2 * Ho * Wo * 4,
    )
    out = pl.pallas_call(
        body,
        out_shape=jax.ShapeDtypeStruct((N, C2, Ho * Wo), jnp.float32),
        grid=(N // B,),
        in_specs=[
            pl.BlockSpec((B, RI, C1, 2 * W), lambda i: (i, 0, 0, 0)),
            pl.BlockSpec((C2, KKC1), lambda i: (0, 0)),
        ],
        out_specs=pl.BlockSpec((B, C2, Ho * Wo), lambda i: (i, 0, 0)),
        scratch_shapes=[
            pltpu.VMEM((RI, C1, 256), jnp.bfloat16),
            pltpu.VMEM((4, C1, RQ * CQ), jnp.bfloat16),
            pltpu.VMEM((KKC1, B * span), jnp.bfloat16),
        ],
        compiler_params=pltpu.CompilerParams(
            dimension_semantics=("parallel",),
            vmem_limit_bytes=64 << 20,
        ),
        cost_estimate=cost,
    )(x_sr, w2)

    return out.reshape(N, C2, Ho, Wo)


def kernel(x_nchw, weight_oihw):
    return _cov_act(x_nchw, weight_oihw, k=3, s=2, p=1, imgs_per_step=2)


# speedup vs baseline: 2.0983x; 2.0983x over previous
import functools

import jax
import jax.numpy as jnp
from jax.experimental import pallas as pl
from jax.experimental.pallas import tpu as pltpu


def _conv_silu_kernel(x_hbm, w_ref, o_ref, xs_ref, eo_ref, p_ref, a_ref,
                      sem, *, k, c1, ri, cq, span, ho, wo):
    i = pl.program_id(0)
    lanes = xs_ref.shape[2]
    ncols = lanes // 2
    rq = ri + 2

    for r in range(ri):
        pltpu.make_async_copy(
            x_hbm.at[i, :, r, :], xs_ref.at[r], sem).start()

    l_io = jax.lax.broadcasted_iota(jnp.int32, (lanes, 256), 0)
    o_io = jax.lax.broadcasted_iota(jnp.int32, (lanes, 256), 1)
    q_o = o_io // cq
    s_o = o_io % cq - 1
    l_want = (q_o // 2) * ncols + 2 * s_o + (q_o % 2)
    valid = (o_io < 4 * cq) & (o_io % cq >= 1) & (s_o < ncols // 2)
    sel = jnp.where(valid & (l_io == l_want), 1.0, 0.0).astype(jnp.bfloat16)

    p_ref[:, :, 0:cq] = jnp.zeros((4, c1, cq), jnp.bfloat16)
    p_ref[:, :, (rq - 1) * cq:rq * cq] = jnp.zeros((4, c1, cq), jnp.bfloat16)

    pltpu.make_async_copy(xs_ref, xs_ref, sem).wait()

    eo = jnp.dot(xs_ref[...].astype(jnp.bfloat16).reshape(ri * c1, lanes),
                 sel, preferred_element_type=jnp.float32)
    eo_ref[...] = eo.astype(jnp.bfloat16).reshape(ri, c1, 256)

    for r in range(ri):
        for q in range(4):
            p_ref[q, :, (1 + r) * cq:(2 + r) * cq] = (
                eo_ref[r, :, q * cq:(q + 1) * cq])

    for kh in range(k):
        for kw in range(k):
            t = kh * k + kw
            q = ((kh + 1) % 2) * 2 + (kw + 1) % 2
            off = (0 if kh == 0 else 1) * cq + (0 if kw == 0 else 1)
            a_ref[t * c1:(t + 1) * c1, :] = p_ref[q, :, off:off + span]

    acc = jnp.dot(w_ref[...], a_ref[...], preferred_element_type=jnp.float32)
    acc = acc * jax.nn.sigmoid(acc)
    for r in range(ho):
        o_ref[0, :, r * wo:(r + 1) * wo] = acc[:, r * cq:r * cq + wo]


@functools.partial(jax.jit, static_argnames=("k", "s", "p"))
def _cov_act(x_nchw, weight_oihw, *, k, s, p):
    N, C1, H, W = x_nchw.shape
    C2 = weight_oihw.shape[0]
    Ho = (H + 2 * p - k) // s + 1
    Wo = (W + 2 * p - k) // s + 1
    KKC1 = k * k * C1
    RI = H // 2
    CQ = W // 2 + 2
    RQ = RI + 2
    span = Ho * CQ

    x_sr = x_nchw.reshape(N, C1, RI, 2 * W)

    w2 = weight_oihw.transpose(0, 2, 3, 1).reshape(C2, KKC1)
    w2 = w2.astype(jnp.bfloat16)

    body = functools.partial(_conv_silu_kernel, k=k, c1=C1, ri=RI, cq=CQ,
                             span=span, ho=Ho, wo=Wo)
    cost = pl.CostEstimate(
        flops=2 * C2 * KKC1 * N * span + 2 * N * RI * C1 * 2 * W * 256,
        transcendentals=N * C2 * span,
        bytes_accessed=N * C1 * H * W * 4 + C2 * KKC1 * 2
        + N * C2 * Ho * Wo * 4,
    )
    out = pl.pallas_call(
        body,
        out_shape=jax.ShapeDtypeStruct((N, C2, Ho * Wo), jnp.float32),
        grid=(N,),
        in_specs=[
            pl.BlockSpec(memory_space=pl.ANY),
            pl.BlockSpec((C2, KKC1), lambda i: (0, 0)),
        ],
        out_specs=pl.BlockSpec((1, C2, Ho * Wo), lambda i: (i, 0, 0)),
        scratch_shapes=[
            pltpu.VMEM((RI, C1, 2 * W), jnp.float32),
            pltpu.VMEM((RI, C1, 256), jnp.bfloat16),
            pltpu.VMEM((4, C1, RQ * CQ), jnp.bfloat16),
            pltpu.VMEM((KKC1, span), jnp.bfloat16),
            pltpu.SemaphoreType.DMA,
        ],
        compiler_params=pltpu.CompilerParams(
            dimension_semantics=("parallel",),
            vmem_limit_bytes=64 << 20,
        ),
        cost_estimate=cost,
    )(x_sr, w2)

    return out.reshape(N, C2, Ho, Wo)


def kernel(x_nchw, weight_oihw):
    return _cov_act(x_nchw, weight_oihw, k=3, s=2, p=1)


# speedup vs baseline: 2.9747x; 1.4177x over previous
import functools

import jax
import jax.numpy as jnp
from jax.experimental import pallas as pl
from jax.experimental.pallas import tpu as pltpu


def _conv_silu_kernel(x_hbm, w_ref, o_ref, xs_ref, eo_ref, p_ref, a_ref,
                      sem, *, k, c1, ri, cq, span, ho, wo, w_in):
    i = pl.program_id(0)
    rq = ri + 2

    for r in range(ri):
        for h in range(2):
            pltpu.make_async_copy(
                x_hbm.at[i, :, 2 * r + h, :], xs_ref.at[r, h], sem).start()

    l_io = jax.lax.broadcasted_iota(jnp.int32, (w_in, 128), 0)
    o_io = jax.lax.broadcasted_iota(jnp.int32, (w_in, 128), 1)
    b_o = o_io // cq
    s_o = o_io % cq - 1
    l_want = 2 * s_o + b_o
    valid = (o_io < 2 * cq) & (o_io % cq >= 1) & (s_o < w_in // 2)
    sel = jnp.where(valid & (l_io == l_want), 1.0, 0.0).astype(jnp.bfloat16)

    p_ref[:, :, 0:cq] = jnp.zeros((4, c1, cq), jnp.bfloat16)
    p_ref[:, :, (rq - 1) * cq:rq * cq] = jnp.zeros((4, c1, cq), jnp.bfloat16)

    pltpu.make_async_copy(xs_ref, xs_ref, sem).wait()

    eo = jnp.dot(xs_ref[...].astype(jnp.bfloat16).reshape(ri * 2 * c1, w_in),
                 sel, preferred_element_type=jnp.float32)
    eo_ref[...] = eo.astype(jnp.bfloat16).reshape(ri, 2, c1, 128)

    for r in range(ri):
        for a in range(2):
            for b1 in range(2):
                p_ref[2 * a + b1, :, (1 + r) * cq:(2 + r) * cq] = (
                    eo_ref[r, a, :, b1 * cq:(b1 + 1) * cq])

    for kh in range(k):
        for kw in range(k):
            t = kh * k + kw
            q = ((kh + 1) % 2) * 2 + (kw + 1) % 2
            off = (0 if kh == 0 else 1) * cq + (0 if kw == 0 else 1)
            a_ref[t * c1:(t + 1) * c1, :] = p_ref[q, :, off:off + span]

    acc = jnp.dot(w_ref[...], a_ref[...], preferred_element_type=jnp.float32)
    acc = acc * jax.nn.sigmoid(acc)
    for r in range(ho):
        o_ref[0, :, r * wo:(r + 1) * wo] = acc[:, r * cq:r * cq + wo]


@functools.partial(jax.jit, static_argnames=("k", "s", "p"))
def _cov_act(x_nchw, weight_oihw, *, k, s, p):
    N, C1, H, W = x_nchw.shape
    C2 = weight_oihw.shape[0]
    Ho = (H + 2 * p - k) // s + 1
    Wo = (W + 2 * p - k) // s + 1
    KKC1 = k * k * C1
    RI = H // 2
    CQ = W // 2 + 2
    RQ = RI + 2
    span = Ho * CQ

    w2 = weight_oihw.transpose(0, 2, 3, 1).reshape(C2, KKC1)
    w2 = w2.astype(jnp.bfloat16)

    body = functools.partial(_conv_silu_kernel, k=k, c1=C1, ri=RI, cq=CQ,
                             span=span, ho=Ho, wo=Wo, w_in=W)
    cost = pl.CostEstimate(
        flops=2 * C2 * KKC1 * N * span + 2 * N * RI * 2 * C1 * 128 * 128,
        transcendentals=N * C2 * span,
        bytes_accessed=N * C1 * H * W * 4 + C2 * KKC1 * 2
        + N * C2 * Ho * Wo * 4,
    )
    out = pl.pallas_call(
        body,
        out_shape=jax.ShapeDtypeStruct((N, C2, Ho * Wo), jnp.float32),
        grid=(N,),
        in_specs=[
            pl.BlockSpec(memory_space=pl.ANY),
            pl.BlockSpec((C2, KKC1), lambda i: (0, 0)),
        ],
        out_specs=pl.BlockSpec((1, C2, Ho * Wo), lambda i: (i, 0, 0)),
        scratch_shapes=[
            pltpu.VMEM((RI, 2, C1, W), jnp.float32),
            pltpu.VMEM((RI, 2, C1, 128), jnp.bfloat16),
            pltpu.VMEM((4, C1, RQ * CQ), jnp.bfloat16),
            pltpu.VMEM((KKC1, span), jnp.bfloat16),
            pltpu.SemaphoreType.DMA,
        ],
        compiler_params=pltpu.CompilerParams(
            dimension_semantics=("parallel",),
            vmem_limit_bytes=64 << 20,
        ),
        cost_estimate=cost,
    )(x_nchw, w2)

    return out.reshape(N, C2, Ho, Wo)


def kernel(x_nchw, weight_oihw):
    return _cov_act(x_nchw, weight_oihw, k=3, s=2, p=1)


# speedup vs baseline: 3.9283x; 1.3206x over previous
import functools

import jax
import jax.numpy as jnp
from jax.experimental import pallas as pl
from jax.experimental.pallas import tpu as pltpu


def _conv_silu_kernel(x_hbm, w_ref, o_ref, xs_ref, eo_ref, p_ref, a_ref,
                      sem, *, k, c1, ri, cq, span, ho, wo, w_in):
    i = pl.program_id(0)
    n_imgs = pl.num_programs(0)
    rq = ri + 2
    slot = jax.lax.rem(i, 2)

    def issue(img, sl):
        for r in range(ri):
            for h in range(2):
                pltpu.make_async_copy(
                    x_hbm.at[img, :, 2 * r + h, :],
                    xs_ref.at[sl, r, h], sem.at[sl]).start()

    @pl.when(i == 0)
    def _():
        issue(i, slot)

    @pl.when(i + 1 < n_imgs)
    def _():
        issue(i + 1, 1 - slot)

    l_io = jax.lax.broadcasted_iota(jnp.int32, (w_in, 128), 0)
    o_io = jax.lax.broadcasted_iota(jnp.int32, (w_in, 128), 1)
    b_o = o_io // cq
    s_o = o_io % cq - 1
    l_want = 2 * s_o + b_o
    valid = (o_io < 2 * cq) & (o_io % cq >= 1) & (s_o < w_in // 2)
    sel = jnp.where(valid & (l_io == l_want), 1.0, 0.0).astype(jnp.bfloat16)

    p_ref[:, :, 0:cq] = jnp.zeros((4, c1, cq), jnp.bfloat16)
    p_ref[:, :, (rq - 1) * cq:rq * cq] = jnp.zeros((4, c1, cq), jnp.bfloat16)

    pltpu.make_async_copy(xs_ref.at[slot], xs_ref.at[slot],
                          sem.at[slot]).wait()

    eo = jnp.dot(
        xs_ref[slot].astype(jnp.bfloat16).reshape(ri * 2 * c1, w_in),
        sel, preferred_element_type=jnp.float32)
    eo_ref[...] = eo.astype(jnp.bfloat16).reshape(ri, 2, c1, 128)

    for r in range(ri):
        for a in range(2):
            for b1 in range(2):
                p_ref[2 * a + b1, :, (1 + r) * cq:(2 + r) * cq] = (
                    eo_ref[r, a, :, b1 * cq:(b1 + 1) * cq])

    for kh in range(k):
        for kw in range(k):
            t = kh * k + kw
            q = ((kh + 1) % 2) * 2 + (kw + 1) % 2
            off = (0 if kh == 0 else 1) * cq + (0 if kw == 0 else 1)
            a_ref[t * c1:(t + 1) * c1, :] = p_ref[q, :, off:off + span]

    acc = jnp.dot(w_ref[...], a_ref[...], preferred_element_type=jnp.float32)
    acc = acc * jax.nn.sigmoid(acc)
    for r in range(ho):
        o_ref[0, :, r * wo:(r + 1) * wo] = acc[:, r * cq:r * cq + wo]


@functools.partial(jax.jit, static_argnames=("k", "s", "p"))
def _cov_act(x_nchw, weight_oihw, *, k, s, p):
    N, C1, H, W = x_nchw.shape
    C2 = weight_oihw.shape[0]
    Ho = (H + 2 * p - k) // s + 1
    Wo = (W + 2 * p - k) // s + 1
    KKC1 = k * k * C1
    RI = H // 2
    CQ = W // 2 + 2
    RQ = RI + 2
    span = Ho * CQ

    w2 = weight_oihw.transpose(0, 2, 3, 1).reshape(C2, KKC1)
    w2 = w2.astype(jnp.bfloat16)

    body = functools.partial(_conv_silu_kernel, k=k, c1=C1, ri=RI, cq=CQ,
                             span=span, ho=Ho, wo=Wo, w_in=W)
    cost = pl.CostEstimate(
        flops=2 * C2 * KKC1 * N * span + 2 * N * RI * 2 * C1 * 128 * 128,
        transcendentals=N * C2 * span,
        bytes_accessed=N * C1 * H * W * 4 + C2 * KKC1 * 2
        + N * C2 * Ho * Wo * 4,
    )
    out = pl.pallas_call(
        body,
        out_shape=jax.ShapeDtypeStruct((N, C2, Ho * Wo), jnp.float32),
        grid=(N,),
        in_specs=[
            pl.BlockSpec(memory_space=pl.ANY),
            pl.BlockSpec((C2, KKC1), lambda i: (0, 0)),
        ],
        out_specs=pl.BlockSpec((1, C2, Ho * Wo), lambda i: (i, 0, 0)),
        scratch_shapes=[
            pltpu.VMEM((2, RI, 2, C1, W), jnp.float32),
            pltpu.VMEM((RI, 2, C1, 128), jnp.bfloat16),
            pltpu.VMEM((4, C1, RQ * CQ), jnp.bfloat16),
            pltpu.VMEM((KKC1, span), jnp.bfloat16),
            pltpu.SemaphoreType.DMA((2,)),
        ],
        compiler_params=pltpu.CompilerParams(
            dimension_semantics=("arbitrary",),
            vmem_limit_bytes=64 << 20,
        ),
        cost_estimate=cost,
    )(x_nchw, w2)

    return out.reshape(N, C2, Ho, Wo)


def kernel(x_nchw, weight_oihw):
    return _cov_act(x_nchw, weight_oihw, k=3, s=2, p=1)


# speedup vs baseline: 4.6511x; 1.1840x over previous
import functools

import jax
import jax.numpy as jnp
from jax.experimental import pallas as pl
from jax.experimental.pallas import tpu as pltpu


def _conv_silu_kernel(x_hbm, w_ref, o_ref, xs_ref, eo_ref, p_ref, a_ref,
                      sem, *, k, c1, ri, cq, span, ho, wo, w_in):
    i = pl.program_id(0)
    n_imgs = pl.num_programs(0)
    rq = ri + 2
    slot = jax.lax.rem(i, 2)

    def issue(img, sl):
        for r in range(ri):
            for h in range(2):
                pltpu.make_async_copy(
                    x_hbm.at[img, :, 2 * r + h, :],
                    xs_ref.at[sl, r, h], sem.at[sl]).start()

    @pl.when(i == 0)
    def _():
        issue(i, slot)

    @pl.when(i + 1 < n_imgs)
    def _():
        issue(i + 1, 1 - slot)

    l_io = jax.lax.broadcasted_iota(jnp.int32, (w_in, 128), 0)
    o_io = jax.lax.broadcasted_iota(jnp.int32, (w_in, 128), 1)
    b_o = o_io // cq
    s_o = o_io % cq - 1
    l_want = 2 * s_o + b_o
    valid = (o_io < 2 * cq) & (o_io % cq >= 1) & (s_o < w_in // 2)
    sel = jnp.where(valid & (l_io == l_want), 1.0, 0.0).astype(jnp.bfloat16)

    p_ref[:, :, 0:cq] = jnp.zeros((4, c1, cq), jnp.bfloat16)
    p_ref[:, :, (rq - 1) * cq:rq * cq] = jnp.zeros((4, c1, cq), jnp.bfloat16)

    pltpu.make_async_copy(xs_ref.at[slot], xs_ref.at[slot],
                          sem.at[slot]).wait()

    eo = jnp.dot(
        xs_ref[slot].astype(jnp.bfloat16).reshape(ri * 2 * c1, w_in),
        sel, preferred_element_type=jnp.float32)
    eo_ref[...] = eo.astype(jnp.bfloat16).reshape(ri, 2, c1, 128)

    for r in range(ri):
        for a in range(2):
            for b1 in range(2):
                p_ref[2 * a + b1, :, (1 + r) * cq:(2 + r) * cq] = (
                    eo_ref[r, a, :, b1 * cq:(b1 + 1) * cq])

    for kh in range(k):
        for kw in range(k):
            t = kh * k + kw
            q = ((kh + 1) % 2) * 2 + (kw + 1) % 2
            off = (0 if kh == 0 else 1) * cq + (0 if kw == 0 else 1)
            a_ref[t * c1:(t + 1) * c1, :] = p_ref[q, :, off:off + span]

    acc = jnp.dot(w_ref[...], a_ref[...], preferred_element_type=jnp.float32)
    acc = acc * jax.nn.sigmoid(acc)
    for r in range(ho):
        o_ref[0, :, r * wo:(r + 1) * wo] = acc[:, r * cq:r * cq + wo]


@functools.partial(jax.jit, static_argnames=("k", "s", "p"))
def _cov_act(x_nchw, weight_oihw, *, k, s, p):
    N, C1, H, W = x_nchw.shape
    C2 = weight_oihw.shape[0]
    Ho = (H + 2 * p - k) // s + 1
    Wo = (W + 2 * p - k) // s + 1
    KKC1 = k * k * C1
    RI = H // 2
    CQ = max(W // 2 + 2, 64)
    RQ = RI + 2
    span = Ho * CQ

    w2 = weight_oihw.transpose(0, 2, 3, 1).reshape(C2, KKC1)
    w2 = w2.astype(jnp.bfloat16)

    body = functools.partial(_conv_silu_kernel, k=k, c1=C1, ri=RI, cq=CQ,
                             span=span, ho=Ho, wo=Wo, w_in=W)
    cost = pl.CostEstimate(
        flops=2 * C2 * KKC1 * N * span + 2 * N * RI * 2 * C1 * 128 * 128,
        transcendentals=N * C2 * span,
        bytes_accessed=N * C1 * H * W * 4 + C2 * KKC1 * 2
        + N * C2 * Ho * Wo * 4,
    )
    out = pl.pallas_call(
        body,
        out_shape=jax.ShapeDtypeStruct((N, C2, Ho * Wo), jnp.float32),
        grid=(N,),
        in_specs=[
            pl.BlockSpec(memory_space=pl.ANY),
            pl.BlockSpec((C2, KKC1), lambda i: (0, 0)),
        ],
        out_specs=pl.BlockSpec((1, C2, Ho * Wo), lambda i: (i, 0, 0)),
        scratch_shapes=[
            pltpu.VMEM((2, RI, 2, C1, W), jnp.float32),
            pltpu.VMEM((RI, 2, C1, 128), jnp.bfloat16),
            pltpu.VMEM((4, C1, RQ * CQ), jnp.bfloat16),
            pltpu.VMEM((KKC1, span), jnp.bfloat16),
            pltpu.SemaphoreType.DMA((2,)),
        ],
        compiler_params=pltpu.CompilerParams(
            dimension_semantics=("arbitrary",),
            vmem_limit_bytes=64 << 20,
        ),
        cost_estimate=cost,
    )(x_nchw, w2)

    return out.reshape(N, C2, Ho, Wo)


def kernel(x_nchw, weight_oihw):
    return _cov_act(x_nchw, weight_oihw, k=3, s=2, p=1)


# speedup vs baseline: 4.6734x; 1.0048x over previous
import functools

import jax
import jax.numpy as jnp
from jax.experimental import pallas as pl
from jax.experimental.pallas import tpu as pltpu


def _conv_silu_kernel(x_hbm, w_ref, o_ref, xs_ref, eo_ref, p_ref, a_ref,
                      sem, *, k, c1, ri, cq, span, ho, wo, w_in, b):
    i = pl.program_id(0)
    n_steps = pl.num_programs(0)
    rq = ri + 2
    slot = jax.lax.rem(i, 2)

    def issue(step, sl):
        for n in range(b):
            for r in range(ri):
                for h in range(2):
                    pltpu.make_async_copy(
                        x_hbm.at[step * b + n, :, 2 * r + h, :],
                        xs_ref.at[sl, n, r, h], sem.at[sl]).start()

    @pl.when(i == 0)
    def _():
        issue(i, slot)

    @pl.when(i + 1 < n_steps)
    def _():
        issue(i + 1, 1 - slot)

    l_io = jax.lax.broadcasted_iota(jnp.int32, (w_in, 128), 0)
    o_io = jax.lax.broadcasted_iota(jnp.int32, (w_in, 128), 1)
    b_o = o_io // cq
    s_o = o_io % cq - 1
    l_want = 2 * s_o + b_o
    valid = (o_io < 2 * cq) & (o_io % cq >= 1) & (s_o < w_in // 2)
    sel = jnp.where(valid & (l_io == l_want), 1.0, 0.0).astype(jnp.bfloat16)

    p_ref[:, :, 0:cq] = jnp.zeros((4, c1, cq), jnp.bfloat16)
    p_ref[:, :, (rq - 1) * cq:rq * cq] = jnp.zeros((4, c1, cq), jnp.bfloat16)

    pltpu.make_async_copy(xs_ref.at[slot], xs_ref.at[slot],
                          sem.at[slot]).wait()

    for n in range(b):
        eo = jnp.dot(
            xs_ref[slot, n].astype(jnp.bfloat16).reshape(ri * 2 * c1, w_in),
            sel, preferred_element_type=jnp.float32)
        eo_ref[...] = eo.astype(jnp.bfloat16).reshape(ri, 2, c1, 128)

        for r in range(ri):
            for a in range(2):
                for b1 in range(2):
                    p_ref[2 * a + b1, :, (1 + r) * cq:(2 + r) * cq] = (
                        eo_ref[r, a, :, b1 * cq:(b1 + 1) * cq])

        for kh in range(k):
            for kw in range(k):
                t = kh * k + kw
                q = ((kh + 1) % 2) * 2 + (kw + 1) % 2
                off = (0 if kh == 0 else 1) * cq + (0 if kw == 0 else 1)
                a_ref[t * c1:(t + 1) * c1, n * span:(n + 1) * span] = (
                    p_ref[q, :, off:off + span])

    acc = jnp.dot(w_ref[...], a_ref[...], preferred_element_type=jnp.float32)
    acc = acc * jax.nn.sigmoid(acc)
    for n in range(b):
        for r in range(ho):
            o_ref[n, :, r * wo:(r + 1) * wo] = (
                acc[:, n * span + r * cq:n * span + r * cq + wo])


@functools.partial(jax.jit, static_argnames=("k", "s", "p"))
def _cov_act(x_nchw, weight_oihw, *, k, s, p):
    N, C1, H, W = x_nchw.shape
    C2 = weight_oihw.shape[0]
    Ho = (H + 2 * p - k) // s + 1
    Wo = (W + 2 * p - k) // s + 1
    KKC1 = k * k * C1
    RI = H // 2
    CQ = max(W // 2 + 2, 64)
    RQ = RI + 2
    span = Ho * CQ

    w2 = weight_oihw.transpose(0, 2, 3, 1).reshape(C2, KKC1)
    w2 = w2.astype(jnp.bfloat16)

    B = 2 if N % 2 == 0 else 1
    body = functools.partial(_conv_silu_kernel, k=k, c1=C1, ri=RI, cq=CQ,
                             span=span, ho=Ho, wo=Wo, w_in=W, b=B)
    cost = pl.CostEstimate(
        flops=2 * C2 * KKC1 * N * span + 2 * N * RI * 2 * C1 * 128 * 128,
        transcendentals=N * C2 * span,
        bytes_accessed=N * C1 * H * W * 4 + C2 * KKC1 * 2
        + N * C2 * Ho * Wo * 4,
    )
    out = pl.pallas_call(
        body,
        out_shape=jax.ShapeDtypeStruct((N, C2, Ho * Wo), jnp.float32),
        grid=(N // B,),
        in_specs=[
            pl.BlockSpec(memory_space=pl.ANY),
            pl.BlockSpec((C2, KKC1), lambda i: (0, 0)),
        ],
        out_specs=pl.BlockSpec((B, C2, Ho * Wo), lambda i: (i, 0, 0)),
        scratch_shapes=[
            pltpu.VMEM((2, B, RI, 2, C1, W), jnp.float32),
            pltpu.VMEM((RI, 2, C1, 128), jnp.bfloat16),
            pltpu.VMEM((4, C1, RQ * CQ), jnp.bfloat16),
            pltpu.VMEM((KKC1, B * span), jnp.bfloat16),
            pltpu.SemaphoreType.DMA((2,)),
        ],
        compiler_params=pltpu.CompilerParams(
            dimension_semantics=("arbitrary",),
            vmem_limit_bytes=64 << 20,
        ),
        cost_estimate=cost,
    )(x_nchw, w2)

    return out.reshape(N, C2, Ho, Wo)


def kernel(x_nchw, weight_oihw):
    return _cov_act(x_nchw, weight_oihw, k=3, s=2, p=1)


# speedup vs baseline: 4.6823x; 1.0019x over previous
import functools

import jax
import jax.numpy as jnp
from jax.experimental import pallas as pl
from jax.experimental.pallas import tpu as pltpu


def _conv_silu_kernel(x_hbm, w_ref, o_ref, xs_ref, eo_ref, p_ref, a_ref,
                      sem, *, k, c1, ri, cq, span, ho, wo, w_in, b):
    i = pl.program_id(0)
    n_steps = pl.num_programs(0)
    rq = ri + 2
    slot = jax.lax.rem(i, 2)

    def issue(step, sl):
        for n in range(b):
            for r in range(ri):
                for h in range(2):
                    pltpu.make_async_copy(
                        x_hbm.at[step * b + n, :, 2 * r + h, :],
                        xs_ref.at[sl, n, r, h], sem.at[sl]).start()

    @pl.when(i == 0)
    def _():
        issue(i, slot)

    @pl.when(i + 1 < n_steps)
    def _():
        issue(i + 1, 1 - slot)

    l_io = jax.lax.broadcasted_iota(jnp.int32, (w_in, 128), 0)
    o_io = jax.lax.broadcasted_iota(jnp.int32, (w_in, 128), 1)
    b_o = o_io // cq
    s_o = o_io % cq - 1
    l_want = 2 * s_o + b_o
    valid = (o_io < 2 * cq) & (o_io % cq >= 1) & (s_o < w_in // 2)
    sel = jnp.where(valid & (l_io == l_want), 1.0, 0.0).astype(jnp.bfloat16)

    p_ref[:, :, 0:cq] = jnp.zeros((4, c1, cq), jnp.bfloat16)
    p_ref[:, :, (rq - 1) * cq:rq * cq] = jnp.zeros((4, c1, cq), jnp.bfloat16)

    pltpu.make_async_copy(xs_ref.at[slot], xs_ref.at[slot],
                          sem.at[slot]).wait()

    for n in range(b):
        eo = jnp.dot(
            xs_ref[slot, n].astype(jnp.bfloat16).reshape(ri * 2 * c1, w_in),
            sel, preferred_element_type=jnp.float32)
        eo_ref[...] = eo.astype(jnp.bfloat16).reshape(ri, 2, c1, 128)

        for r in range(ri):
            for a in range(2):
                for b1 in range(2):
                    p_ref[2 * a + b1, :, (1 + r) * cq:(2 + r) * cq] = (
                        eo_ref[r, a, :, b1 * cq:(b1 + 1) * cq])

        for kh in range(k):
            for kw in range(k):
                t = kh * k + kw
                q = ((kh + 1) % 2) * 2 + (kw + 1) % 2
                off = (0 if kh == 0 else 1) * cq + (0 if kw == 0 else 1)
                a_ref[t * c1:(t + 1) * c1, n * span:(n + 1) * span] = (
                    p_ref[q, :, off:off + span])

    acc = jnp.dot(w_ref[...], a_ref[...], preferred_element_type=jnp.float32)
    for n in range(b):
        for r in range(ho):
            v = acc[:, n * span + r * cq:n * span + r * cq + wo]
            o_ref[n, :, r * wo:(r + 1) * wo] = v * jax.nn.sigmoid(v)


@functools.partial(jax.jit, static_argnames=("k", "s", "p"))
def _cov_act(x_nchw, weight_oihw, *, k, s, p):
    N, C1, H, W = x_nchw.shape
    C2 = weight_oihw.shape[0]
    Ho = (H + 2 * p - k) // s + 1
    Wo = (W + 2 * p - k) // s + 1
    KKC1 = k * k * C1
    RI = H // 2
    CQ = max(W // 2 + 2, 64)
    RQ = RI + 2
    span = Ho * CQ

    w2 = weight_oihw.transpose(0, 2, 3, 1).reshape(C2, KKC1)
    w2 = w2.astype(jnp.bfloat16)

    B = 2 if N % 2 == 0 else 1
    body = functools.partial(_conv_silu_kernel, k=k, c1=C1, ri=RI, cq=CQ,
                             span=span, ho=Ho, wo=Wo, w_in=W, b=B)
    cost = pl.CostEstimate(
        flops=2 * C2 * KKC1 * N * span + 2 * N * RI * 2 * C1 * 128 * 128,
        transcendentals=N * C2 * span,
        bytes_accessed=N * C1 * H * W * 4 + C2 * KKC1 * 2
        + N * C2 * Ho * Wo * 4,
    )
    out = pl.pallas_call(
        body,
        out_shape=jax.ShapeDtypeStruct((N, C2, Ho * Wo), jnp.float32),
        grid=(N // B,),
        in_specs=[
            pl.BlockSpec(memory_space=pl.ANY),
            pl.BlockSpec((C2, KKC1), lambda i: (0, 0)),
        ],
        out_specs=pl.BlockSpec((B, C2, Ho * Wo), lambda i: (i, 0, 0)),
        scratch_shapes=[
            pltpu.VMEM((2, B, RI, 2, C1, W), jnp.float32),
            pltpu.VMEM((RI, 2, C1, 128), jnp.bfloat16),
            pltpu.VMEM((4, C1, RQ * CQ), jnp.bfloat16),
            pltpu.VMEM((KKC1, B * span), jnp.bfloat16),
            pltpu.SemaphoreType.DMA((2,)),
        ],
        compiler_params=pltpu.CompilerParams(
            dimension_semantics=("arbitrary",),
            vmem_limit_bytes=64 << 20,
        ),
        cost_estimate=cost,
    )(x_nchw, w2)

    return out.reshape(N, C2, Ho, Wo)


def kernel(x_nchw, weight_oihw):
    return _cov_act(x_nchw, weight_oihw, k=3, s=2, p=1)
